# Initial kernel scaffold; baseline (speedup 1.0000x reference)
import jax
import jax.numpy as jnp
from jax.experimental import pallas as pl


def kernel(embedding, pos_table, l0_wqkv, l0_bqkv, l0_wo, l0_bo, l0_w1, l0_b1, l0_w2, l0_b2, l0_ln1_g, l0_ln1_b, l0_ln2_g, l0_ln2_b, l1_wqkv, l1_bqkv, l1_wo, l1_bo, l1_w1, l1_b1, l1_w2, l1_b2, l1_ln1_g, l1_ln1_b, l1_ln2_g, l1_ln2_b, inputs):
    raise NotImplementedError("write your pallas kernel here")


# speedup vs baseline: 1.1137x; 1.1137x over previous
import functools
import math

import jax
import jax.numpy as jnp
from jax.experimental import pallas as pl
from jax.experimental.pallas import tpu as pltpu


def _layer(x, mask, wqkv, bqkv, wo, bo, w1, b1, w2, b2, g1, be1, g2, be2,
           *, n_heads, scale, eps):
    s, d = x.shape
    dk = d // n_heads

    xb = x.astype(jnp.bfloat16)
    qkv = jnp.dot(xb, wqkv, preferred_element_type=jnp.float32) + bqkv

    outs = []
    for h in range(n_heads):
        q = qkv[:, h * dk:(h + 1) * dk].astype(jnp.bfloat16)
        k = qkv[:, d + h * dk:d + (h + 1) * dk].astype(jnp.bfloat16)
        v = qkv[:, 2 * d + h * dk:2 * d + (h + 1) * dk].astype(jnp.bfloat16)
        sc = jax.lax.dot_general(q, k, (((1,), (1,)), ((), ())),
                                 preferred_element_type=jnp.float32) * scale
        sc = jnp.where(mask > 0.5, jnp.float32(-1e9), sc)
        mx = jnp.max(sc, axis=-1, keepdims=True)
        p = jnp.exp(sc - mx)
        l = jnp.sum(p, axis=-1, keepdims=True)
        w = (p * pl.reciprocal(l)).astype(jnp.bfloat16)
        outs.append(jnp.dot(w, v, preferred_element_type=jnp.float32))
    attn = jnp.concatenate(outs, axis=-1).astype(jnp.bfloat16)

    y = jnp.dot(attn, wo, preferred_element_type=jnp.float32) + bo + x
    mean = jnp.mean(y, axis=-1, keepdims=True)
    yc = y - mean
    var = jnp.mean(yc * yc, axis=-1, keepdims=True)
    h1 = yc * jax.lax.rsqrt(var + eps) * g1 + be1

    f = jnp.dot(h1.astype(jnp.bfloat16), w1, preferred_element_type=jnp.float32) + b1
    f = jnp.maximum(f, 0.0).astype(jnp.bfloat16)
    f = jnp.dot(f, w2, preferred_element_type=jnp.float32) + b2
    y2 = h1 + f
    mean2 = jnp.mean(y2, axis=-1, keepdims=True)
    yc2 = y2 - mean2
    var2 = jnp.mean(yc2 * yc2, axis=-1, keepdims=True)
    return yc2 * jax.lax.rsqrt(var2 + eps) * g2 + be2


def _encoder_kernel(x_ref, m_ref, *refs, n_heads, scale, eps):
    o_ref = refs[-1]
    wrefs = refs[:-1]
    x = x_ref[...]
    mask = m_ref[...]
    for li in range(2):
        w = [r[...] for r in wrefs[li * 12:(li + 1) * 12]]
        x = _layer(x, mask, *w, n_heads=n_heads, scale=scale, eps=eps)
    o_ref[...] = x


def kernel(embedding, pos_table,
           l0_wqkv, l0_bqkv, l0_wo, l0_bo, l0_w1, l0_b1, l0_w2, l0_b2,
           l0_ln1_g, l0_ln1_b, l0_ln2_g, l0_ln2_b,
           l1_wqkv, l1_bqkv, l1_wo, l1_bo, l1_w1, l1_b1, l1_w2, l1_b2,
           l1_ln1_g, l1_ln1_b, l1_ln2_g, l1_ln2_b,
           inputs):
    b, s = inputs.shape
    d = embedding.shape[1]
    ff = l0_w1.shape[1]
    n_heads = 8
    pad_id = 0
    scale = 1.0 / math.sqrt(d // n_heads)

    positions = jnp.broadcast_to(jnp.arange(1, s + 1, dtype=jnp.int32), (b, s))
    positions = jnp.where(inputs == pad_id, 0, positions)
    x = jnp.take(embedding, inputs, axis=0) + jnp.take(pos_table, positions, axis=0)
    mask3 = (inputs == pad_id).astype(jnp.float32).reshape(b, 1, s)

    bf = jnp.bfloat16
    weights = [
        l0_wqkv.astype(bf), l0_bqkv, l0_wo.astype(bf), l0_bo,
        l0_w1.astype(bf), l0_b1, l0_w2.astype(bf), l0_b2,
        l0_ln1_g, l0_ln1_b, l0_ln2_g, l0_ln2_b,
        l1_wqkv.astype(bf), l1_bqkv, l1_wo.astype(bf), l1_bo,
        l1_w1.astype(bf), l1_b1, l1_w2.astype(bf), l1_b2,
        l1_ln1_g, l1_ln1_b, l1_ln2_g, l1_ln2_b,
    ]

    def const_spec(arr):
        nd = arr.ndim
        return pl.BlockSpec(arr.shape, lambda i: (0,) * nd)

    in_specs = [
        pl.BlockSpec((None, s, d), lambda i: (i, 0, 0)),
        pl.BlockSpec((None, 1, s), lambda i: (i, 0, 0)),
    ] + [const_spec(w) for w in weights]

    fn = functools.partial(_encoder_kernel, n_heads=n_heads, scale=scale,
                           eps=1e-6)
    out = pl.pallas_call(
        fn,
        out_shape=jax.ShapeDtypeStruct((b, s, d), jnp.float32),
        grid=(b,),
        in_specs=in_specs,
        out_specs=pl.BlockSpec((None, s, d), lambda i: (i, 0, 0)),
        compiler_params=pltpu.CompilerParams(dimension_semantics=("parallel",)),
    )(x, mask3, *weights)
    return out


# speedup vs baseline: 1.2701x; 1.1405x over previous
import functools
import math

import jax
import jax.numpy as jnp
from jax.experimental import pallas as pl
from jax.experimental.pallas import tpu as pltpu

_SEQ_PER_STEP = 4


def _layer(x, masks, wqkv, bqkv, wo, bo, w1, b1, w2, b2, g1, be1, g2, be2,
           *, n_heads, scale, eps, s):
    n, d = x.shape
    dk = d // n_heads

    xb = x.astype(jnp.bfloat16)
    qkv = jnp.dot(xb, wqkv, preferred_element_type=jnp.float32) + bqkv

    attn_parts = []
    for si in range(n // s):
        mask = masks[si]
        base = si * s
        outs = []
        for h in range(n_heads):
            q = qkv[base:base + s, h * dk:(h + 1) * dk].astype(jnp.bfloat16)
            k = qkv[base:base + s, d + h * dk:d + (h + 1) * dk].astype(jnp.bfloat16)
            v = qkv[base:base + s, 2 * d + h * dk:2 * d + (h + 1) * dk].astype(jnp.bfloat16)
            sc = jax.lax.dot_general(q, k, (((1,), (1,)), ((), ())),
                                     preferred_element_type=jnp.float32) * scale
            sc = jnp.where(mask > 0.5, jnp.float32(-1e9), sc)
            mx = jnp.max(sc, axis=-1, keepdims=True)
            p = jnp.exp(sc - mx)
            l = jnp.sum(p, axis=-1, keepdims=True)
            w = (p * pl.reciprocal(l)).astype(jnp.bfloat16)
            outs.append(jnp.dot(w, v, preferred_element_type=jnp.float32))
        attn_parts.append(jnp.concatenate(outs, axis=-1))
    attn = jnp.concatenate(attn_parts, axis=0).astype(jnp.bfloat16)

    y = jnp.dot(attn, wo, preferred_element_type=jnp.float32) + bo + x
    mean = jnp.mean(y, axis=-1, keepdims=True)
    yc = y - mean
    var = jnp.mean(yc * yc, axis=-1, keepdims=True)
    h1 = yc * jax.lax.rsqrt(var + eps) * g1 + be1

    f = jnp.dot(h1.astype(jnp.bfloat16), w1, preferred_element_type=jnp.float32) + b1
    f = jnp.maximum(f, 0.0).astype(jnp.bfloat16)
    f = jnp.dot(f, w2, preferred_element_type=jnp.float32) + b2
    y2 = h1 + f
    mean2 = jnp.mean(y2, axis=-1, keepdims=True)
    yc2 = y2 - mean2
    var2 = jnp.mean(yc2 * yc2, axis=-1, keepdims=True)
    return yc2 * jax.lax.rsqrt(var2 + eps) * g2 + be2


def _encoder_kernel(x_ref, m_ref, pos_ref, pos0_ref, *refs, n_heads, scale, eps):
    o_ref = refs[-1]
    wrefs = refs[:-1]
    ns, s, d = x_ref.shape
    pos = pos_ref[...]
    pos0 = pos0_ref[...]
    masks = [m_ref[si] for si in range(ns)]
    xs = []
    for si in range(ns):
        m_col = masks[si].reshape(s, 1)
        xs.append(x_ref[si] + jnp.where(m_col > 0.5, pos0, pos))
    x = jnp.concatenate(xs, axis=0)
    for li in range(2):
        w = [r[...] for r in wrefs[li * 12:(li + 1) * 12]]
        x = _layer(x, masks, *w, n_heads=n_heads, scale=scale, eps=eps, s=s)
    o_ref[...] = x.reshape(ns, s, d)


def kernel(embedding, pos_table,
           l0_wqkv, l0_bqkv, l0_wo, l0_bo, l0_w1, l0_b1, l0_w2, l0_b2,
           l0_ln1_g, l0_ln1_b, l0_ln2_g, l0_ln2_b,
           l1_wqkv, l1_bqkv, l1_wo, l1_bo, l1_w1, l1_b1, l1_w2, l1_b2,
           l1_ln1_g, l1_ln1_b, l1_ln2_g, l1_ln2_b,
           inputs):
    b, s = inputs.shape
    d = embedding.shape[1]
    n_heads = 8
    pad_id = 0
    scale = 1.0 / math.sqrt(d // n_heads)
    nseq = _SEQ_PER_STEP

    x_tok = jnp.take(embedding, inputs, axis=0)
    mask3 = (inputs == pad_id).astype(jnp.float32).reshape(b, 1, s)
    pos_rows = jax.lax.slice(pos_table, (1, 0), (s + 1, d))
    pos0 = jax.lax.slice(pos_table, (0, 0), (1, d))

    bf = jnp.bfloat16
    weights = [
        l0_wqkv.astype(bf), l0_bqkv, l0_wo.astype(bf), l0_bo,
        l0_w1.astype(bf), l0_b1, l0_w2.astype(bf), l0_b2,
        l0_ln1_g, l0_ln1_b, l0_ln2_g, l0_ln2_b,
        l1_wqkv.astype(bf), l1_bqkv, l1_wo.astype(bf), l1_bo,
        l1_w1.astype(bf), l1_b1, l1_w2.astype(bf), l1_b2,
        l1_ln1_g, l1_ln1_b, l1_ln2_g, l1_ln2_b,
    ]

    def const_spec(arr):
        nd = arr.ndim
        return pl.BlockSpec(arr.shape, lambda i: (0,) * nd)

    in_specs = [
        pl.BlockSpec((nseq, s, d), lambda i: (i, 0, 0)),
        pl.BlockSpec((nseq, 1, s), lambda i: (i, 0, 0)),
        const_spec(pos_rows),
        const_spec(pos0),
    ] + [const_spec(w) for w in weights]

    fn = functools.partial(_encoder_kernel, n_heads=n_heads, scale=scale,
                           eps=1e-6)
    out = pl.pallas_call(
        fn,
        out_shape=jax.ShapeDtypeStruct((b, s, d), jnp.float32),
        grid=(b // nseq,),
        in_specs=in_specs,
        out_specs=pl.BlockSpec((nseq, s, d), lambda i: (i, 0, 0)),
        compiler_params=pltpu.CompilerParams(dimension_semantics=("parallel",)),
    )(x_tok, mask3, pos_rows, pos0, *weights)
    return out


# speedup vs baseline: 1.6044x; 1.2631x over previous
import functools
import math

import jax
import jax.numpy as jnp
from jax.experimental import pallas as pl
from jax.experimental.pallas import tpu as pltpu

_SEQ_PER_STEP = 4


def _layer(x, mbias, wqkv, bqkv, wo, bo, w1, b1, w2, b2, g1, be1, g2, be2,
           *, n_heads, eps, s):
    n, d = x.shape
    dk = d // n_heads

    xb = x.astype(jnp.bfloat16)
    qkv = jnp.dot(xb, wqkv, preferred_element_type=jnp.float32) + bqkv

    attn_parts = []
    for si in range(n // s):
        bias = mbias[si]
        base = si * s
        outs = []
        for h in range(n_heads):
            q = qkv[base:base + s, h * dk:(h + 1) * dk].astype(jnp.bfloat16)
            k = qkv[base:base + s, d + h * dk:d + (h + 1) * dk].astype(jnp.bfloat16)
            va = qkv[base:base + s, 2 * d + h * 2 * dk:2 * d + (h + 1) * 2 * dk]
            sc = jax.lax.dot_general(q, k, (((1,), (1,)), ((), ())),
                                     preferred_element_type=jnp.float32)
            p = jnp.exp(jnp.minimum(sc + bias, 60.0)).astype(jnp.bfloat16)
            oa = jnp.dot(p, va.astype(jnp.bfloat16),
                         preferred_element_type=jnp.float32)
            l = oa[:, dk:dk + 1]
            outs.append(oa[:, :dk] * pl.reciprocal(l))
        attn_parts.append(jnp.concatenate(outs, axis=-1))
    attn = jnp.concatenate(attn_parts, axis=0).astype(jnp.bfloat16)

    y = jnp.dot(attn, wo, preferred_element_type=jnp.float32) + bo + x
    mean = jnp.mean(y, axis=-1, keepdims=True)
    yc = y - mean
    var = jnp.mean(yc * yc, axis=-1, keepdims=True)
    h1 = yc * jax.lax.rsqrt(var + eps) * g1 + be1

    f = jnp.dot(h1.astype(jnp.bfloat16), w1, preferred_element_type=jnp.float32) + b1
    f = jnp.maximum(f, 0.0).astype(jnp.bfloat16)
    f = jnp.dot(f, w2, preferred_element_type=jnp.float32) + b2
    y2 = h1 + f
    mean2 = jnp.mean(y2, axis=-1, keepdims=True)
    yc2 = y2 - mean2
    var2 = jnp.mean(yc2 * yc2, axis=-1, keepdims=True)
    return yc2 * jax.lax.rsqrt(var2 + eps) * g2 + be2


def _encoder_kernel(x_ref, mb_ref, mc_ref, pos_ref, pos0_ref, *refs,
                    n_heads, eps):
    o_ref = refs[-1]
    wrefs = refs[:-1]
    ns, s, d = x_ref.shape
    pos = pos_ref[...]
    pos0 = pos0_ref[...]
    mcol = mc_ref[...]
    x = (x_ref[...] + jnp.where(mcol > 0.5, pos0, pos)).reshape(ns * s, d)
    mbias = [mb_ref[si] for si in range(ns)]
    for li in range(2):
        w = [r[...] for r in wrefs[li * 12:(li + 1) * 12]]
        x = _layer(x, mbias, *w, n_heads=n_heads, eps=eps, s=s)
    o_ref[...] = x.reshape(ns, s, d)


def _prep_qkv(wqkv, bqkv, d, n_heads, scale):
    dk = d // n_heads
    wq = wqkv[:, :d] * scale
    wk = wqkv[:, d:2 * d]
    wv = wqkv[:, 2 * d:].reshape(d, n_heads, dk)
    wv_aug = jnp.concatenate([wv, jnp.zeros_like(wv)], axis=2).reshape(d, 2 * d)
    bq = bqkv[:, :d] * scale
    bk = bqkv[:, d:2 * d]
    bv = bqkv[:, 2 * d:].reshape(1, n_heads, dk)
    ones_col = jnp.zeros((1, n_heads, dk), jnp.float32).at[:, :, 0].set(1.0)
    bv_aug = jnp.concatenate([bv, ones_col], axis=2).reshape(1, 2 * d)
    w = jnp.concatenate([wq, wk, wv_aug], axis=1).astype(jnp.bfloat16)
    b = jnp.concatenate([bq, bk, bv_aug], axis=1)
    return w, b


def kernel(embedding, pos_table,
           l0_wqkv, l0_bqkv, l0_wo, l0_bo, l0_w1, l0_b1, l0_w2, l0_b2,
           l0_ln1_g, l0_ln1_b, l0_ln2_g, l0_ln2_b,
           l1_wqkv, l1_bqkv, l1_wo, l1_bo, l1_w1, l1_b1, l1_w2, l1_b2,
           l1_ln1_g, l1_ln1_b, l1_ln2_g, l1_ln2_b,
           inputs):
    b, s = inputs.shape
    d = embedding.shape[1]
    n_heads = 8
    pad_id = 0
    scale = 1.0 / math.sqrt(d // n_heads)
    nseq = _SEQ_PER_STEP

    x_tok = jnp.take(embedding, inputs, axis=0)
    padm = (inputs == pad_id)
    mbias = jnp.where(padm, jnp.float32(-1e9), 0.0).reshape(b, 1, s)
    mcol = padm.astype(jnp.float32).reshape(b, s, 1)
    pos_rows = jax.lax.slice(pos_table, (1, 0), (s + 1, d)).reshape(1, s, d)
    pos0 = jax.lax.slice(pos_table, (0, 0), (1, d)).reshape(1, 1, d)

    bf = jnp.bfloat16
    w0qkv, b0qkv = _prep_qkv(l0_wqkv, l0_bqkv, d, n_heads, scale)
    w1qkv, b1qkv = _prep_qkv(l1_wqkv, l1_bqkv, d, n_heads, scale)
    weights = [
        w0qkv, b0qkv, l0_wo.astype(bf), l0_bo,
        l0_w1.astype(bf), l0_b1, l0_w2.astype(bf), l0_b2,
        l0_ln1_g, l0_ln1_b, l0_ln2_g, l0_ln2_b,
        w1qkv, b1qkv, l1_wo.astype(bf), l1_bo,
        l1_w1.astype(bf), l1_b1, l1_w2.astype(bf), l1_b2,
        l1_ln1_g, l1_ln1_b, l1_ln2_g, l1_ln2_b,
    ]

    def const_spec(arr):
        nd = arr.ndim
        return pl.BlockSpec(arr.shape, lambda i: (0,) * nd)

    in_specs = [
        pl.BlockSpec((nseq, s, d), lambda i: (i, 0, 0)),
        pl.BlockSpec((nseq, 1, s), lambda i: (i, 0, 0)),
        pl.BlockSpec((nseq, s, 1), lambda i: (i, 0, 0)),
        const_spec(pos_rows),
        const_spec(pos0),
    ] + [const_spec(w) for w in weights]

    fn = functools.partial(_encoder_kernel, n_heads=n_heads, eps=1e-6)
    out = pl.pallas_call(
        fn,
        out_shape=jax.ShapeDtypeStruct((b, s, d), jnp.float32),
        grid=(b // nseq,),
        in_specs=in_specs,
        out_specs=pl.BlockSpec((nseq, s, d), lambda i: (i, 0, 0)),
        compiler_params=pltpu.CompilerParams(dimension_semantics=("parallel",)),
    )(x_tok, mbias, mcol, pos_rows, pos0, *weights)
    return out
